# baseline (speedup 1.0000x reference)
import math

import jax
import jax.numpy as jnp
from jax.experimental import pallas as pl
from jax.experimental.pallas import tpu as pltpu

_INV_SQRT2 = 1.0 / math.sqrt(2.0)


def _mask_body(wu_ref, mu_ref, wd_ref, md_ref, wum_ref, wdm_ref):
    wum_ref[...] = (wu_ref[...] * mu_ref[...].astype(jnp.float32)).astype(
        jnp.bfloat16
    )
    wdm_ref[...] = (wd_ref[...] * md_ref[...].astype(jnp.float32)).astype(
        jnp.bfloat16
    )


def _up_body(x_ref, wu_ref, g_ref, h_acc):
    k = pl.program_id(2)
    nk = pl.num_programs(2)

    part = jax.lax.dot_general(
        x_ref[...].astype(jnp.bfloat16),
        wu_ref[...],
        (((1,), (1,)), ((), ())),
        preferred_element_type=jnp.float32,
    )

    @pl.when(k == 0)
    def _():
        h_acc[...] = part

    @pl.when(k != 0)
    def _():
        h_acc[...] += part

    @pl.when(k == nk - 1)
    def _():
        rows = h_acc.shape[0]
        chunk = min(256, rows)

        def body(i, carry):
            h = h_acc[pl.ds(i * chunk, chunk), :]
            g = 0.5 * h * (1.0 + jax.lax.erf(h * _INV_SQRT2))
            g_ref[pl.ds(i * chunk, chunk), :] = g.astype(jnp.bfloat16)
            return carry

        jax.lax.fori_loop(0, rows // chunk, body, 0)


def _down_body(g_ref, wd_ref, out_ref):
    f = pl.program_id(1)

    o = jax.lax.dot_general(
        g_ref[...],
        wd_ref[...],
        (((1,), (1,)), ((), ())),
        preferred_element_type=jnp.float32,
    )

    @pl.when(f == 0)
    def _():
        out_ref[...] = o

    @pl.when(f != 0)
    def _():
        out_ref[...] += o


@jax.jit
def kernel(ffn_input_cat, W_up, W_down, mask_up, mask_down):
    tok, d_in = ffn_input_cat.shape
    d_ff = W_up.shape[0]
    d_model = W_down.shape[0]

    mu8 = mask_up.astype(jnp.int8)
    md8 = mask_down.astype(jnp.int8)

    pb = min(256, d_ff)
    steps = d_ff // pb
    pd = d_model // steps
    wum, wdm = pl.pallas_call(
        _mask_body,
        grid=(steps,),
        in_specs=[
            pl.BlockSpec((pb, d_in), lambda i: (i, 0)),
            pl.BlockSpec((pb, d_in), lambda i: (i, 0)),
            pl.BlockSpec((pd, d_ff), lambda i: (i, 0)),
            pl.BlockSpec((pd, d_ff), lambda i: (i, 0)),
        ],
        out_specs=[
            pl.BlockSpec((pb, d_in), lambda i: (i, 0)),
            pl.BlockSpec((pd, d_ff), lambda i: (i, 0)),
        ],
        out_shape=[
            jax.ShapeDtypeStruct((d_ff, d_in), jnp.bfloat16),
            jax.ShapeDtypeStruct((d_model, d_ff), jnp.bfloat16),
        ],
    )(W_up, mu8, W_down, md8)

    tb = min(1024, tok)
    fb = min(2048, d_ff)
    kb = min(2048, d_in)
    g = pl.pallas_call(
        _up_body,
        grid=(d_ff // fb, tok // tb, d_in // kb),
        in_specs=[
            pl.BlockSpec((tb, kb), lambda f, t, k: (t, k)),
            pl.BlockSpec((fb, kb), lambda f, t, k: (f, k)),
        ],
        out_specs=pl.BlockSpec((tb, fb), lambda f, t, k: (t, f)),
        out_shape=jax.ShapeDtypeStruct((tok, d_ff), jnp.bfloat16),
        scratch_shapes=[pltpu.VMEM((tb, fb), jnp.float32)],
    )(ffn_input_cat, wum)

    tb2 = min(2048, tok)
    fb2 = min(1024, d_ff)
    out = pl.pallas_call(
        _down_body,
        grid=(tok // tb2, d_ff // fb2),
        in_specs=[
            pl.BlockSpec((tb2, fb2), lambda t, f: (t, f)),
            pl.BlockSpec((d_model, fb2), lambda t, f: (0, f)),
        ],
        out_specs=pl.BlockSpec((tb2, d_model), lambda t, f: (t, 0)),
        out_shape=jax.ShapeDtypeStruct((tok, d_model), jnp.float32),
    )(g, wdm)
    return out


# speedup vs baseline: 1.0181x; 1.0181x over previous
import math

import jax
import jax.numpy as jnp
from jax.experimental import pallas as pl
from jax.experimental.pallas import tpu as pltpu

_INV_SQRT2 = 1.0 / math.sqrt(2.0)


def _mask_body(w_ref, m_ref, out_ref):
    out_ref[...] = (w_ref[...] * m_ref[...].astype(jnp.float32)).astype(jnp.bfloat16)


def _up_body(x_ref, wu_ref, g_ref, h_acc):
    k = pl.program_id(2)
    nk = pl.num_programs(2)

    part = jax.lax.dot_general(
        x_ref[...].astype(jnp.bfloat16),
        wu_ref[...],
        (((1,), (1,)), ((), ())),
        preferred_element_type=jnp.float32,
    )

    @pl.when(k == 0)
    def _():
        h_acc[...] = part

    @pl.when(k != 0)
    def _():
        h_acc[...] += part

    @pl.when(k == nk - 1)
    def _():
        rows = h_acc.shape[0]
        chunk = min(256, rows)

        def body(i, carry):
            h = h_acc[pl.ds(i * chunk, chunk), :]
            g = 0.5 * h * (1.0 + jax.lax.erf(h * _INV_SQRT2))
            g_ref[pl.ds(i * chunk, chunk), :] = g.astype(jnp.bfloat16)
            return carry

        jax.lax.fori_loop(0, rows // chunk, body, 0)


def _down_body(g_ref, wd_ref, md_ref, out_ref):
    f = pl.program_id(1)

    wdb = wd_ref[...].astype(jnp.bfloat16) * md_ref[...].astype(jnp.bfloat16)
    o = jax.lax.dot_general(
        g_ref[...], wdb, (((1,), (1,)), ((), ())), preferred_element_type=jnp.float32
    )

    @pl.when(f == 0)
    def _():
        out_ref[...] = o

    @pl.when(f != 0)
    def _():
        out_ref[...] += o


@jax.jit
def kernel(ffn_input_cat, W_up, W_down, mask_up, mask_down):
    tok, d_in = ffn_input_cat.shape
    d_ff = W_up.shape[0]
    d_model = W_down.shape[0]

    mu8 = mask_up.astype(jnp.int8)
    md8 = mask_down.astype(jnp.int8)

    pb = min(256, d_ff)
    wum = pl.pallas_call(
        _mask_body,
        grid=(d_ff // pb,),
        in_specs=[
            pl.BlockSpec((pb, d_in), lambda i: (i, 0)),
            pl.BlockSpec((pb, d_in), lambda i: (i, 0)),
        ],
        out_specs=pl.BlockSpec((pb, d_in), lambda i: (i, 0)),
        out_shape=jax.ShapeDtypeStruct((d_ff, d_in), jnp.bfloat16),
    )(W_up, mu8)

    tb = min(1024, tok)
    fb = min(2048, d_ff)
    kb = min(2048, d_in)
    g = pl.pallas_call(
        _up_body,
        grid=(d_ff // fb, tok // tb, d_in // kb),
        in_specs=[
            pl.BlockSpec((tb, kb), lambda f, t, k: (t, k)),
            pl.BlockSpec((fb, kb), lambda f, t, k: (f, k)),
        ],
        out_specs=pl.BlockSpec((tb, fb), lambda f, t, k: (t, f)),
        out_shape=jax.ShapeDtypeStruct((tok, d_ff), jnp.bfloat16),
        scratch_shapes=[pltpu.VMEM((tb, fb), jnp.float32)],
    )(ffn_input_cat, wum)

    tb2 = min(2048, tok)
    fb2 = min(1024, d_ff)
    out = pl.pallas_call(
        _down_body,
        grid=(tok // tb2, d_ff // fb2),
        in_specs=[
            pl.BlockSpec((tb2, fb2), lambda t, f: (t, f)),
            pl.BlockSpec((d_model, fb2), lambda t, f: (0, f)),
            pl.BlockSpec((d_model, fb2), lambda t, f: (0, f)),
        ],
        out_specs=pl.BlockSpec((tb2, d_model), lambda t, f: (t, 0)),
        out_shape=jax.ShapeDtypeStruct((tok, d_model), jnp.float32),
    )(g, W_down, md8)
    return out
